# baseline (speedup 1.0000x reference)
import jax
import jax.numpy as jnp
from jax import lax
from jax.experimental import pallas as pl
from jax.experimental.pallas import tpu as pltpu
from jax.experimental.pallas import tpu_sc as plsc

L = 16
NC, NS = 2, 16
NW = NC * NS
ROWS, COLS = 16384, 1024
ROWS_W = ROWS // NW
CR = 16
NCHUNK = ROWS_W // CR
NSTEP = NCHUNK // 2
UNROLL = 8
H, W = 5, 9


def _body(phi_hbm, sq_hbm, tab_hbm, consts_hbm, out_hbm,
          tab_v, consts_v, phi_v0, phi_v1, sq_v0, sq_v1, out_v0, out_v1,
          sem_in0, sem_in1, sem_out0, sem_out1):
    wid = lax.axis_index("s") * NC + lax.axis_index("c")
    row0 = wid * ROWS_W

    pltpu.sync_copy(tab_hbm, tab_v)
    pltpu.sync_copy(consts_hbm, consts_v)
    biasv = consts_v[pl.ds(0, L)]
    scalev = consts_v[pl.ds(L, L)]
    t_a = tab_v.at[0]
    t_bx = tab_v.at[1]
    t_by = tab_v.at[2]
    t_bxy = tab_v.at[3]

    phi_v = (phi_v0, phi_v1)
    sq_v = (sq_v0, sq_v1)
    out_v = (out_v0, out_v1)
    sem_in = (sem_in0, sem_in1)
    sem_out = (sem_out0, sem_out1)

    def in_slices(i):
        r = row0 + i * CR
        return phi_hbm.at[pl.ds(r, CR)], sq_hbm.at[pl.ds(r, CR)]

    def out_slice(i):
        return out_hbm.at[pl.ds(row0 + i * CR, CR)]

    def start_in(i, b):
        ps, ss = in_slices(i)
        pltpu.async_copy(ps, phi_v[b], sem_in[b])
        pltpu.async_copy(ss, sq_v[b], sem_in[b])

    def wait_in(i, b):
        ps, ss = in_slices(i)
        pltpu.make_async_copy(ps, phi_v[b], sem_in[b]).wait()
        pltpu.make_async_copy(ss, sq_v[b], sem_in[b]).wait()

    def interp(p, q):
        r = p - p.astype(jnp.int32).astype(jnp.float32)
        a = jnp.abs(r)
        pe = jnp.minimum(a, 1.0 - a)
        x = pe * jnp.float32(2 * (W - 1))
        y = jnp.clip(q * scalev + biasv, 0.0, jnp.float32(H - 1))
        x0 = jnp.minimum(x.astype(jnp.int32), W - 2)
        y0 = jnp.minimum(y.astype(jnp.int32), H - 2)
        fx = x - x0.astype(jnp.float32)
        fy = y - y0.astype(jnp.float32)
        idx = y0 * (W - 1) + x0
        ca = plsc.load_gather(t_a, [idx])
        cbx = plsc.load_gather(t_bx, [idx])
        cby = plsc.load_gather(t_by, [idx])
        cbxy = plsc.load_gather(t_bxy, [idx])
        return ca + cbx * fx + cby * fy + cbxy * (fx * fy)

    def compute(b):
        pv, sv, ov = phi_v[b], sq_v[b], out_v[b]

        @plsc.parallel_loop(0, CR * COLS, step=L, unroll=UNROLL)
        def _(i):
            rr = lax.shift_right_logical(i, 10)
            cc = lax.bitwise_and(i, COLS - 1)
            sl = pl.ds(cc, L)
            ov[rr, sl] = interp(pv[rr, sl], sv[rr, sl])

    start_in(0, 0)
    start_in(1, 1)

    def step_body(s, carry):
        for b in (0, 1):
            i = s * 2 + b
            wait_in(i, b)

            @pl.when(s > 0)
            def _():
                pltpu.make_async_copy(out_v[b], out_slice(i - 2),
                                      sem_out[b]).wait()

            compute(b)
            pltpu.async_copy(out_v[b], out_slice(i), sem_out[b])

            @pl.when(s < NSTEP - 1)
            def _():
                start_in(i + 2, b)

        return carry

    lax.fori_loop(0, NSTEP, step_body, 0)

    for b in (0, 1):
        pltpu.make_async_copy(out_v[b], out_slice(NCHUNK - 2 + b),
                              sem_out[b]).wait()


@jax.jit
def _run(phi, sq, tab, consts):
    mesh = plsc.VectorSubcoreMesh(core_axis_name="c", subcore_axis_name="s")
    return pl.kernel(
        _body,
        out_type=jax.ShapeDtypeStruct((ROWS, COLS), jnp.float32),
        mesh=mesh,
        compiler_params=pltpu.CompilerParams(needs_layout_passes=False),
        scratch_types=[
            pltpu.VMEM((4, (H - 1) * (W - 1)), jnp.float32),
            pltpu.VMEM((2 * L,), jnp.float32),
            pltpu.VMEM((CR, COLS), jnp.float32),
            pltpu.VMEM((CR, COLS), jnp.float32),
            pltpu.VMEM((CR, COLS), jnp.float32),
            pltpu.VMEM((CR, COLS), jnp.float32),
            pltpu.VMEM((CR, COLS), jnp.float32),
            pltpu.VMEM((CR, COLS), jnp.float32),
            pltpu.SemaphoreType.DMA,
            pltpu.SemaphoreType.DMA,
            pltpu.SemaphoreType.DMA,
            pltpu.SemaphoreType.DMA,
        ],
    )(phi, sq, tab, consts)


def kernel(phi, squid_current, g_table, ib_list):
    g = g_table
    c_a = g[:H - 1, :W - 1]
    c_bx = g[:H - 1, 1:] - c_a
    c_by = g[1:, :W - 1] - c_a
    c_bxy = g[1:, 1:] - g[1:, :W - 1] - g[:H - 1, 1:] + c_a
    tab = jnp.stack([c_a.reshape(-1), c_bx.reshape(-1),
                     c_by.reshape(-1), c_bxy.reshape(-1)])
    scale = jnp.float32(H - 1) / (ib_list[-1] - ib_list[0])
    bias = -ib_list[0] * scale
    consts = jnp.concatenate([
        jnp.full((L,), bias, jnp.float32),
        jnp.full((L,), scale, jnp.float32),
    ])
    return _run(phi, squid_current, tab, consts)


# speedup vs baseline: 1.0967x; 1.0967x over previous
import jax
import jax.numpy as jnp
from jax import lax
from jax.experimental import pallas as pl
from jax.experimental.pallas import tpu as pltpu
from jax.experimental.pallas import tpu_sc as plsc

L = 16
NC, NS = 2, 16
NW = NC * NS
ROWS, COLS = 16384, 1024
ROWS_W = ROWS // NW
CR = 16
NCHUNK = ROWS_W // CR
NSTEP = NCHUNK // 2
UNROLL = 4
H, W = 5, 9


def _body(phi_hbm, sq_hbm, tab_hbm, consts_hbm, out_hbm,
          tab_v, consts_v, phi_v0, phi_v1, sq_v0, sq_v1, out_v0, out_v1,
          sem_in0, sem_in1, sem_out0, sem_out1):
    wid = lax.axis_index("s") * NC + lax.axis_index("c")
    row0 = wid * ROWS_W

    pltpu.sync_copy(tab_hbm, tab_v)
    pltpu.sync_copy(consts_hbm, consts_v)
    biasv = consts_v[pl.ds(0, L)]
    scalev = consts_v[pl.ds(L, L)]
    t_a = tab_v.at[0]
    t_bx = tab_v.at[1]
    t_by = tab_v.at[2]
    t_bxy = tab_v.at[3]

    phi_v = (phi_v0, phi_v1)
    sq_v = (sq_v0, sq_v1)
    out_v = (out_v0, out_v1)
    sem_in = (sem_in0, sem_in1)
    sem_out = (sem_out0, sem_out1)

    def in_slices(i):
        r = row0 + i * CR
        return phi_hbm.at[pl.ds(r, CR)], sq_hbm.at[pl.ds(r, CR)]

    def out_slice(i):
        return out_hbm.at[pl.ds(row0 + i * CR, CR)]

    def start_in(i, b):
        ps, ss = in_slices(i)
        pltpu.async_copy(ps, phi_v[b], sem_in[b])
        pltpu.async_copy(ss, sq_v[b], sem_in[b])

    def wait_in(i, b):
        ps, ss = in_slices(i)
        pltpu.make_async_copy(ps, phi_v[b], sem_in[b]).wait()
        pltpu.make_async_copy(ss, sq_v[b], sem_in[b]).wait()

    def interp(p, q):
        r = p - p.astype(jnp.int32).astype(jnp.float32)
        a = jnp.abs(r)
        pe = jnp.minimum(a, 1.0 - a)
        x = pe * jnp.float32(2 * (W - 1))
        y = jnp.clip(q * scalev + biasv, 0.0, jnp.float32(H - 1))
        x0 = jnp.minimum(x.astype(jnp.int32), W - 2)
        y0 = jnp.minimum(y.astype(jnp.int32), H - 2)
        idx = y0 * (W - 1) + x0
        ca = plsc.load_gather(t_a, [idx])
        cbx = plsc.load_gather(t_bx, [idx])
        cby = plsc.load_gather(t_by, [idx])
        cbxy = plsc.load_gather(t_bxy, [idx])
        return ca + cbx * x + cby * y + cbxy * (x * y)

    def compute(b):
        pv, sv, ov = phi_v[b], sq_v[b], out_v[b]

        @plsc.parallel_loop(0, CR * COLS, step=L, unroll=UNROLL)
        def _(i):
            rr = lax.shift_right_logical(i, 10)
            cc = lax.bitwise_and(i, COLS - 1)
            sl = pl.ds(cc, L)
            ov[rr, sl] = interp(pv[rr, sl], sv[rr, sl])

    start_in(0, 0)
    start_in(1, 1)

    def step_body(s, carry):
        for b in (0, 1):
            i = s * 2 + b
            wait_in(i, b)

            @pl.when(s > 0)
            def _():
                pltpu.make_async_copy(out_v[b], out_slice(i - 2),
                                      sem_out[b]).wait()

            compute(b)
            pltpu.async_copy(out_v[b], out_slice(i), sem_out[b])

            @pl.when(s < NSTEP - 1)
            def _():
                start_in(i + 2, b)

        return carry

    lax.fori_loop(0, NSTEP, step_body, 0)

    for b in (0, 1):
        pltpu.make_async_copy(out_v[b], out_slice(NCHUNK - 2 + b),
                              sem_out[b]).wait()


@jax.jit
def _run(phi, sq, tab, consts):
    mesh = plsc.VectorSubcoreMesh(core_axis_name="c", subcore_axis_name="s")
    return pl.kernel(
        _body,
        out_type=jax.ShapeDtypeStruct((ROWS, COLS), jnp.float32),
        mesh=mesh,
        compiler_params=pltpu.CompilerParams(needs_layout_passes=False),
        scratch_types=[
            pltpu.VMEM((4, (H - 1) * (W - 1)), jnp.float32),
            pltpu.VMEM((2 * L,), jnp.float32),
            pltpu.VMEM((CR, COLS), jnp.float32),
            pltpu.VMEM((CR, COLS), jnp.float32),
            pltpu.VMEM((CR, COLS), jnp.float32),
            pltpu.VMEM((CR, COLS), jnp.float32),
            pltpu.VMEM((CR, COLS), jnp.float32),
            pltpu.VMEM((CR, COLS), jnp.float32),
            pltpu.SemaphoreType.DMA,
            pltpu.SemaphoreType.DMA,
            pltpu.SemaphoreType.DMA,
            pltpu.SemaphoreType.DMA,
        ],
    )(phi, sq, tab, consts)


def kernel(phi, squid_current, g_table, ib_list):
    g = g_table
    c_a = g[:H - 1, :W - 1]
    c_bx = g[:H - 1, 1:] - c_a
    c_by = g[1:, :W - 1] - c_a
    c_bxy = g[1:, 1:] - g[1:, :W - 1] - g[:H - 1, 1:] + c_a
    jj = jnp.arange(W - 1, dtype=jnp.float32)[None, :]
    ii = jnp.arange(H - 1, dtype=jnp.float32)[:, None]
    p_a = c_a - c_bx * jj - c_by * ii + c_bxy * ii * jj
    p_bx = c_bx - c_bxy * ii
    p_by = c_by - c_bxy * jj
    tab = jnp.stack([p_a.reshape(-1), p_bx.reshape(-1),
                     p_by.reshape(-1), c_bxy.reshape(-1)])
    scale = jnp.float32(H - 1) / (ib_list[-1] - ib_list[0])
    bias = -ib_list[0] * scale
    consts = jnp.concatenate([
        jnp.full((L,), bias, jnp.float32),
        jnp.full((L,), scale, jnp.float32),
    ])
    return _run(phi, squid_current, tab, consts)
